# baseline (speedup 1.0000x reference)
import functools

import jax
import jax.numpy as jnp
from jax import lax
from jax.experimental import pallas as pl
from jax.experimental.pallas import tpu as pltpu
from jax.experimental.pallas import tpu_sc as plsc


def _make_enc_body(P, BLK, NB, D, OUT, NBLK):
    def body(nf_ref, nbf_ref, W_ref, BD_ref, ESP_ref, DW_ref,
             Ws_ref, bs_ref, q_ref, z_ref, ss_ref, acc_ref):
        i = pl.program_id(0)

        @pl.when(i == 0)
        def _():
            acc_ref[0] = 0.0
            acc_ref[1] = 0.0

        nf = nf_ref[...]
        for p in range(P):
            Wp = W_ref[p]
            x2b = nbf_ref[p].astype(jnp.bfloat16)
            ep = (jnp.dot(nf, ESP_ref[p], preferred_element_type=jnp.float32)
                  + jnp.dot(x2b, BD_ref[p],
                            preferred_element_type=jnp.float32))
            ep = jnp.maximum(ep, 0.2 * ep)
            exq = jnp.exp(ep)
            big = jnp.dot(exq.astype(jnp.bfloat16), DW_ref[...],
                          preferred_element_type=jnp.float32)
            den = big[:, :OUT]
            Wb = Wp.astype(jnp.bfloat16)
            num = None
            for nb in range(NB):
                xnb = x2b[:, nb * D:(nb + 1) * D]
                hbn = jnp.dot(xnb, Wb, preferred_element_type=jnp.float32)
                aw = big[:, (nb + 1) * OUT:(nb + 2) * OUT]
                num = aw * hbn if num is None else num + aw * hbn
            z = num / den
            z = jnp.where(z > 0, z, jnp.exp(jnp.minimum(z, 0.0)) - 1.0)
            z_ref[p] = z
            t = jnp.tanh(jnp.dot(z, Ws_ref[...],
                                 preferred_element_type=jnp.float32)
                         + bs_ref[...])
            acc_ref[p] += jnp.sum(t * q_ref[...])

        @pl.when(i == NBLK - 1)
        def _():
            rows = lax.broadcasted_iota(jnp.int32, (8, 128), 0)
            ss_ref[...] = jnp.where(rows == 0, acc_ref[0], acc_ref[1])

    return body


def _encode(node_feats, nbf3, W, BD, ESP, DW, Ws, bs2, q2, BLK):
    P, B, ND = nbf3.shape
    D = node_feats.shape[1]
    NB = ND // D
    OUT = W.shape[2]
    NH = DW.shape[0]
    NBLK = B // BLK
    return pl.pallas_call(
        _make_enc_body(P, BLK, NB, D, OUT, NBLK),
        grid=(NBLK,),
        in_specs=[
            pl.BlockSpec((BLK, D), lambda i: (i, 0)),
            pl.BlockSpec((P, BLK, ND), lambda i: (0, i, 0)),
            pl.BlockSpec((P, D, OUT), lambda i: (0, 0, 0)),
            pl.BlockSpec((P, ND, NH), lambda i: (0, 0, 0)),
            pl.BlockSpec((P, D, NH), lambda i: (0, 0, 0)),
            pl.BlockSpec((NH, (NB + 1) * OUT), lambda i: (0, 0)),
            pl.BlockSpec((OUT, OUT), lambda i: (0, 0)),
            pl.BlockSpec((1, OUT), lambda i: (0, 0)),
            pl.BlockSpec((1, OUT), lambda i: (0, 0)),
        ],
        out_specs=[
            pl.BlockSpec((P, BLK, OUT), lambda i: (0, i, 0)),
            pl.BlockSpec((8, 128), lambda i: (0, 0)),
        ],
        out_shape=[
            jax.ShapeDtypeStruct((P, B, OUT), jnp.float32),
            jax.ShapeDtypeStruct((8, 128), jnp.float32),
        ],
        scratch_shapes=[pltpu.SMEM((2,), jnp.float32)],
        compiler_params=pltpu.CompilerParams(
            dimension_semantics=("arbitrary",)),
    )(node_feats, nbf3, W, BD, ESP, DW, Ws, bs2, q2)


def _sc_combine(Z, ss):
    P, B, OUT = Z.shape
    CH = 40
    NCHUNK = B // CH
    NC, NS = 2, 16
    NW = NC * NS
    mesh = plsc.VectorSubcoreMesh(core_axis_name="c", subcore_axis_name="s")

    @functools.partial(
        pl.kernel, mesh=mesh,
        out_type=jax.ShapeDtypeStruct((B, OUT), jnp.float32),
        scratch_types=[
            pltpu.VMEM((CH, OUT), jnp.float32),
            pltpu.VMEM((CH, OUT), jnp.float32),
            pltpu.VMEM((2, 128), jnp.float32),
        ],
    )
    def combine(z_hbm, ss_hbm, out_hbm, z0_v, z1_v, s_v):
        wid = lax.axis_index("s") * NC + lax.axis_index("c")
        pltpu.sync_copy(ss_hbm.at[pl.ds(0, 2)], s_v)
        s0 = s_v[0, pl.ds(0, 16)]
        s1 = s_v[1, pl.ds(0, 16)]
        w0 = s0 * (1.0 / B)
        w1 = s1 * (1.0 / B)
        m = jnp.maximum(w0, w1)
        e0 = jnp.exp(w0 - m)
        e1 = jnp.exp(w1 - m)
        rs = 1.0 / (e0 + e1)
        b0 = e0 * rs
        b1 = e1 * rs
        extra = NCHUNK - (NCHUNK // NW) * NW
        nch = jnp.where(wid < extra, NCHUNK // NW + 1, NCHUNK // NW)

        def chunk_body(it, carry):
            base = (wid + it * NW) * CH
            pltpu.sync_copy(z_hbm.at[0, pl.ds(base, CH)], z0_v)
            pltpu.sync_copy(z_hbm.at[1, pl.ds(base, CH)], z1_v)
            for r in range(CH):
                for c in range(OUT // 16):
                    sl = (r, pl.ds(c * 16, 16))
                    z0_v[sl] = b0 * z0_v[sl] + b1 * z1_v[sl]
            pltpu.sync_copy(z0_v, out_hbm.at[pl.ds(base, CH)])
            return carry

        lax.fori_loop(0, nch, chunk_body, 0)

    return combine(Z, ss)


def kernel(src_nodes, labels, node_feats_arr, neighbor_feats_arr,
           W, att_self, att_neigh, Ws, bs, q):
    P, B, NB, D = neighbor_feats_arr.shape
    OUT = W.shape[2]
    HEADS, HID = att_self.shape[1], att_self.shape[2]
    BLK = next(blk for blk in (1000, 512, 400, 256, 200, 128, 80, 40, 16, 8, B)
               if B % blk == 0)

    nbf3 = neighbor_feats_arr.reshape(P, B, NB * D)
    NH = NB * HEADS

    W4 = W.reshape(P, D, HEADS, HID)
    WAn = jnp.einsum("pdhk,phk->pdh", W4, att_neigh)
    WAs = jnp.einsum("pdhk,phk->pdh", W4, att_self)
    eyeNB = jnp.eye(NB, dtype=jnp.float32)
    BD = jnp.stack([jnp.kron(eyeNB, WAn[p]) for p in range(P)])
    ESP = jnp.stack([jnp.tile(WAs[p], (1, NB)) for p in range(P)])
    mm = jnp.arange(NH)
    jj = jnp.arange(OUT)
    headm = (mm % HEADS)[:, None]
    nbm = (mm // HEADS)[:, None]
    headj = (jj // HID)[None, :]
    base = (headm == headj).astype(jnp.float32)
    DW = jnp.concatenate(
        [base] + [base * (nbm == nb) for nb in range(NB)], axis=1)

    Z, ss = _encode(node_feats_arr, nbf3, W, BD.astype(jnp.bfloat16), ESP,
                    DW.astype(jnp.bfloat16), Ws,
                    bs.reshape(1, OUT), q.reshape(1, OUT), BLK)
    return _sc_combine(Z, ss)


# speedup vs baseline: 2.0605x; 2.0605x over previous
import functools

import jax
import jax.numpy as jnp
from jax import lax
from jax.experimental import pallas as pl
from jax.experimental.pallas import tpu as pltpu
from jax.experimental.pallas import tpu_sc as plsc


def _make_enc_body(P, BLK, NB, OUT, NBLK):
    def body(nf_ref, nbf_ref, W_ref, M_ref, N_ref, Ws_ref,
             bs_ref, q_ref, z_ref, ss_ref, acc_ref):
        i = pl.program_id(0)

        @pl.when(i == 0)
        def _():
            acc_ref[0] = 0.0
            acc_ref[1] = 0.0

        nf = nf_ref[...]
        for p in range(P):
            Wp = W_ref[p]
            EAs = jnp.dot(Wp, M_ref[p], preferred_element_type=jnp.float32)
            EAn = jnp.dot(Wp, N_ref[p], preferred_element_type=jnp.float32)
            x = nbf_ref[p]
            es = jnp.dot(nf, EAs, preferred_element_type=jnp.float32)
            en = jnp.dot(x, EAn, preferred_element_type=jnp.float32)
            hb = jnp.dot(x, Wp, preferred_element_type=jnp.float32)
            en3 = en.reshape(BLK, NB, OUT)
            hb3 = hb.reshape(BLK, NB, OUT)
            e = es[:, None, :] + en3
            e = jnp.maximum(e, 0.2 * e)
            ex = jnp.exp(e)
            den = jnp.sum(ex, axis=1)
            num = jnp.sum(ex * hb3, axis=1)
            z = num / den
            z = jnp.where(z > 0, z, jnp.exp(jnp.minimum(z, 0.0)) - 1.0)
            z_ref[p] = z
            t = jnp.tanh(jnp.dot(z, Ws_ref[...],
                                 preferred_element_type=jnp.float32)
                         + bs_ref[...])
            acc_ref[p] += jnp.sum(t * q_ref[...])

        @pl.when(i == NBLK - 1)
        def _():
            rows = lax.broadcasted_iota(jnp.int32, (8, 128), 0)
            ss_ref[...] = jnp.where(rows == 0, acc_ref[0], acc_ref[1])

    return body


def _encode(node_feats, nbf2, W, M, N, Ws, bs2, q2, BLK):
    P, BN, D = nbf2.shape
    B = node_feats.shape[0]
    NB = BN // B
    OUT = W.shape[2]
    NBLK = B // BLK
    return pl.pallas_call(
        _make_enc_body(P, BLK, NB, OUT, NBLK),
        grid=(NBLK,),
        in_specs=[
            pl.BlockSpec((BLK, D), lambda i: (i, 0)),
            pl.BlockSpec((P, BLK * NB, D), lambda i: (0, i, 0)),
            pl.BlockSpec((P, D, OUT), lambda i: (0, 0, 0)),
            pl.BlockSpec((P, OUT, OUT), lambda i: (0, 0, 0)),
            pl.BlockSpec((P, OUT, OUT), lambda i: (0, 0, 0)),
            pl.BlockSpec((OUT, OUT), lambda i: (0, 0)),
            pl.BlockSpec((1, OUT), lambda i: (0, 0)),
            pl.BlockSpec((1, OUT), lambda i: (0, 0)),
        ],
        out_specs=[
            pl.BlockSpec((P, BLK, OUT), lambda i: (0, i, 0)),
            pl.BlockSpec((8, 128), lambda i: (0, 0)),
        ],
        out_shape=[
            jax.ShapeDtypeStruct((P, B, OUT), jnp.float32),
            jax.ShapeDtypeStruct((8, 128), jnp.float32),
        ],
        scratch_shapes=[pltpu.SMEM((2,), jnp.float32)],
        compiler_params=pltpu.CompilerParams(
            dimension_semantics=("arbitrary",)),
    )(node_feats, nbf2, W, M, N, Ws, bs2, q2)


def _sc_combine(Z, ss):
    P, B, OUT = Z.shape
    CH = 80
    NCHUNK = B // CH
    NC, NS = 2, 16
    NW = NC * NS
    mesh = plsc.VectorSubcoreMesh(core_axis_name="c", subcore_axis_name="s")

    @functools.partial(
        pl.kernel, mesh=mesh,
        out_type=jax.ShapeDtypeStruct((B, OUT), jnp.float32),
        scratch_types=[
            pltpu.VMEM((CH, OUT), jnp.float32),
            pltpu.VMEM((CH, OUT), jnp.float32),
            pltpu.VMEM((2, 128), jnp.float32),
            pltpu.SemaphoreType.DMA,
        ],
    )
    def combine(z_hbm, ss_hbm, out_hbm, z0_v, z1_v, s_v, sem):
        wid = lax.axis_index("s") * NC + lax.axis_index("c")
        pltpu.sync_copy(ss_hbm.at[pl.ds(0, 2)], s_v)
        s0 = s_v[0, pl.ds(0, 16)]
        s1 = s_v[1, pl.ds(0, 16)]
        w0 = s0 * (1.0 / B)
        w1 = s1 * (1.0 / B)
        m = jnp.maximum(w0, w1)
        e0 = jnp.exp(w0 - m)
        e1 = jnp.exp(w1 - m)
        rs = 1.0 / (e0 + e1)
        b0 = e0 * rs
        b1 = e1 * rs
        extra = NCHUNK - (NCHUNK // NW) * NW
        nch = jnp.where(wid < extra, NCHUNK // NW + 1, NCHUNK // NW)

        def chunk_body(it, carry):
            base = (wid + it * NW) * CH
            cp0 = pltpu.async_copy(z_hbm.at[0, pl.ds(base, CH)], z0_v, sem)
            cp1 = pltpu.async_copy(z_hbm.at[1, pl.ds(base, CH)], z1_v, sem)
            cp0.wait()
            cp1.wait()
            for r in range(CH):
                for c in range(OUT // 16):
                    sl = (r, pl.ds(c * 16, 16))
                    z0_v[sl] = b0 * z0_v[sl] + b1 * z1_v[sl]
            pltpu.sync_copy(z0_v, out_hbm.at[pl.ds(base, CH)])
            return carry

        lax.fori_loop(0, nch, chunk_body, 0)

    return combine(Z, ss)


def kernel(src_nodes, labels, node_feats_arr, neighbor_feats_arr,
           W, att_self, att_neigh, Ws, bs, q):
    P, B, NB, D = neighbor_feats_arr.shape
    OUT = W.shape[2]
    HEADS, HID = att_self.shape[1], att_self.shape[2]
    BLK = next(blk for blk in (1000, 512, 400, 256, 200, 128, 80, 40, 16, 8, B)
               if B % blk == 0)

    nbf2 = neighbor_feats_arr.reshape(P, B * NB, D)
    eyeH = jnp.eye(HEADS, dtype=jnp.float32)

    def expand(att_p):
        m4 = att_p[:, :, None, None] * eyeH[:, None, :, None]
        m4 = jnp.broadcast_to(m4, (HEADS, HID, HEADS, HID))
        return m4.reshape(OUT, OUT)

    M = jnp.stack([expand(att_self[p]) for p in range(P)])
    N = jnp.stack([expand(att_neigh[p]) for p in range(P)])

    Z, ss = _encode(node_feats_arr, nbf2, W, M, N, Ws,
                    bs.reshape(1, OUT), q.reshape(1, OUT), BLK)
    return _sc_combine(Z, ss)
